# speedup vs baseline: 1.3254x; 1.3254x over previous
import jax
import jax.numpy as jnp
from jax.experimental import pallas as pl

_TB = 256


def _vq_kernel(xt_ref, cb_ref, idx_ref, q_ref):
    xt = xt_ref[...]
    cb = cb_ref[...]
    mm = jax.lax.dot_general(xt, cb, (((1,), (1,)), ((), ())),
                             preferred_element_type=jnp.float32)
    x2 = jnp.sum(xt ** 2, axis=-1, keepdims=True)
    c2 = jnp.sum(cb ** 2, axis=-1)
    dist = x2 + c2[None, :] - 2.0 * mm
    idx = jnp.argmin(dist, axis=1)
    idx_ref[...] = idx
    onehot = (jax.lax.broadcasted_iota(jnp.int32, dist.shape, 1)
              == idx[:, None]).astype(jnp.float32)
    q_ref[...] = jax.lax.dot_general(onehot, cb, (((1,), (0,)), ((), ())),
                                     preferred_element_type=jnp.float32)


def kernel(x, codebook):
    B, D, T = x.shape
    K = codebook.shape[0]
    xt = jnp.transpose(x, (0, 2, 1)).reshape(B * T, D)
    n_blocks = (B * T) // _TB
    idx, q = pl.pallas_call(
        _vq_kernel,
        grid=(n_blocks,),
        in_specs=[pl.BlockSpec((_TB, D), lambda i: (i, 0)),
                  pl.BlockSpec((K, D), lambda i: (0, 0))],
        out_specs=[pl.BlockSpec((_TB,), lambda i: (i,)),
                   pl.BlockSpec((_TB, D), lambda i: (i, 0))],
        out_shape=[jax.ShapeDtypeStruct((B * T,), jnp.int32),
                   jax.ShapeDtypeStruct((B * T, D), jnp.float32)],
    )(xt, codebook)
    quantized = jnp.transpose(q.reshape(B, T, D), (0, 2, 1))
    return quantized, idx.reshape(B, T)
